# baseline (speedup 1.0000x reference)
import functools

import jax
import jax.numpy as jnp
from jax import lax
from jax.experimental import pallas as pl
from jax.experimental.pallas import tpu as pltpu
from jax.experimental.pallas import tpu_sc as plsc

_N = 10000
_E = 160000
_D = 256
_H = 512

_NSUB = 16
_CHUNK = 128
_NCH = 80
_EPS = _NCH * _CHUNK
_EPAD = _NSUB * _EPS
_NA = _N + _NSUB
_RPS = _N // _NSUB


def _sc_aggregate(xv, idxp):
    mesh = plsc.VectorSubcoreMesh(core_axis_name="c", subcore_axis_name="s")

    @functools.partial(
        pl.kernel,
        out_type=jax.ShapeDtypeStruct((2, _N, 128), jnp.float32),
        mesh=mesh,
        scratch_types=[
            pltpu.VMEM((2, _CHUNK), jnp.int32),
            pltpu.VMEM((2, _CHUNK), jnp.int32),
            pltpu.VMEM((_CHUNK, 128), jnp.float32),
            pltpu.VMEM((_CHUNK, 128), jnp.float32),
            pltpu.VMEM((_CHUNK, 16), jnp.float32),
            pltpu.VMEM((_CHUNK, 16), jnp.float32),
            pltpu.VMEM_SHARED((_NA, 128), jnp.float32),
            pltpu.VMEM_SHARED((_NA, 16), jnp.float32),
            pltpu.SemaphoreType.DMA,
            pltpu.SemaphoreType.DMA,
        ],
        compiler_params=pltpu.CompilerParams(use_tc_tiling_on_sc=False),
    )
    def k(xv_hbm, idx_hbm, out_hbm, idx_a, idx_b, buf_a, buf_b, ones_v,
          deg_v, acc, dacc, sem_g, sem_i):
        c = lax.axis_index("c")
        s = lax.axis_index("s")
        idx = idx_hbm.at[s]
        cvec = jnp.full((16,), 0, jnp.int32) + c

        zv = jnp.zeros((1, 16), jnp.float32)
        ov = jnp.ones((1, 16), jnp.float32)

        @pl.loop(0, _CHUNK)
        def _(r):
            for cc in range(0, 128, 16):
                buf_a[pl.ds(r, 1), pl.ds(cc, 16)] = zv
            ones_v[pl.ds(r, 1), pl.ds(0, 16)] = ov
            deg_v[pl.ds(r, 1), pl.ds(0, 16)] = zv

        @pl.loop(0, 5)
        def _(kk):
            ch = s + _NSUB * kk

            @pl.when(ch < _NA // _CHUNK)
            def _():
                pltpu.sync_copy(buf_a, acc.at[pl.ds(ch * _CHUNK, _CHUNK)])
                pltpu.sync_copy(deg_v, dacc.at[pl.ds(ch * _CHUNK, _CHUNK)])

            @pl.when(ch == _NA // _CHUNK)
            def _():
                r0 = ch * _CHUNK
                nr = _NA - r0
                pltpu.sync_copy(buf_a.at[pl.ds(0, nr)], acc.at[pl.ds(r0, nr)])
                pltpu.sync_copy(deg_v.at[pl.ds(0, nr)], dacc.at[pl.ds(r0, nr)])

        plsc.subcore_barrier()

        def load_idx_start(g, ib):
            pltpu.async_copy(idx.at[g], ib, sem_i)

        def load_idx_wait(g, ib):
            pltpu.make_async_copy(idx.at[g], ib, sem_i).wait()
            for cc in range(0, _CHUNK, 16):
                ib[pl.ds(0, 1), pl.ds(cc, 16)] = (
                    ib[pl.ds(0, 1), pl.ds(cc, 16)] + cvec.reshape(1, 16))

        load_idx_start(0, idx_a)
        load_idx_wait(0, idx_a)
        pltpu.async_copy(xv_hbm.at[idx_a.at[0]], buf_a, sem_g)
        load_idx_start(1, idx_b)

        @pl.loop(0, _NCH // 2)
        def _(g2):
            g = 2 * g2
            load_idx_wait(g + 1, idx_b)
            pltpu.make_async_copy(xv_hbm.at[idx_a.at[0]], buf_a, sem_g).wait()
            pltpu.async_copy(xv_hbm.at[idx_b.at[0]], buf_b, sem_g)
            pltpu.sync_copy(buf_a, acc.at[idx_a.at[1]], add=True)
            pltpu.sync_copy(ones_v, dacc.at[idx_a.at[1]], add=True)

            @pl.when(g2 < _NCH // 2 - 1)
            def _():
                load_idx_start(g + 2, idx_a)
                pltpu.make_async_copy(xv_hbm.at[idx_b.at[0]], buf_b, sem_g).wait()
                load_idx_wait(g + 2, idx_a)
                pltpu.async_copy(xv_hbm.at[idx_a.at[0]], buf_a, sem_g)
                pltpu.sync_copy(buf_b, acc.at[idx_b.at[1]], add=True)
                pltpu.sync_copy(ones_v, dacc.at[idx_b.at[1]], add=True)
                load_idx_start(g + 3, idx_b)

            @pl.when(g2 == _NCH // 2 - 1)
            def _():
                pltpu.make_async_copy(xv_hbm.at[idx_b.at[0]], buf_b, sem_g).wait()
                pltpu.sync_copy(buf_b, acc.at[idx_b.at[1]], add=True)
                pltpu.sync_copy(ones_v, dacc.at[idx_b.at[1]], add=True)

        plsc.subcore_barrier()

        @pl.loop(0, 5)
        def _(kk):
            r0 = s * _RPS + kk * 125
            pltpu.sync_copy(acc.at[pl.ds(r0, 125)], buf_a.at[pl.ds(0, 125)])
            pltpu.sync_copy(dacc.at[pl.ds(r0, 125)], deg_v.at[pl.ds(0, 125)])

            @pl.loop(0, 125)
            def _(r):
                ivec = 1.0 / jnp.maximum(deg_v[pl.ds(r, 1), pl.ds(0, 16)], 1.0)

                for cc in range(0, 128, 16):
                    buf_a[pl.ds(r, 1), pl.ds(cc, 16)] = (
                        buf_a[pl.ds(r, 1), pl.ds(cc, 16)] * ivec)

            pltpu.sync_copy(buf_a.at[pl.ds(0, 125)],
                            out_hbm.at[c].at[pl.ds(r0, 125)])

    return k(xv, idxp)


def _mm1_body(x_ref, wr_ref, o_ref):
    o_ref[...] = jnp.dot(x_ref[...], wr_ref[...],
                         preferred_element_type=jnp.float32)


def _mm1(x, W_root):
    B = 2000
    return pl.pallas_call(
        _mm1_body,
        grid=(_N // B,),
        in_specs=[
            pl.BlockSpec((B, _D), lambda i: (i, 0)),
            pl.BlockSpec((_D, _H), lambda i: (0, 0)),
        ],
        out_specs=pl.BlockSpec((B, _H), lambda i: (i, 0)),
        out_shape=jax.ShapeDtypeStruct((_N, _H), jnp.float32),
    )(x, W_root)


def _mlp_body(r_ref, nb_ref, wn_ref, bc_ref, w1_ref, b1_ref,
              w2_ref, b2_ref, w3_ref, b3_ref, o_ref):
    f32 = jnp.float32
    h = r_ref[...]
    h = h + jnp.dot(nb_ref[0], wn_ref[:128], preferred_element_type=f32)
    h = h + jnp.dot(nb_ref[1], wn_ref[128:], preferred_element_type=f32)
    h = jnp.maximum(h + bc_ref[...], 0.0)
    z = jnp.maximum(jnp.dot(h, w1_ref[...], preferred_element_type=f32)
                    + b1_ref[...], 0.0)
    z = jnp.maximum(jnp.dot(z, w2_ref[...], preferred_element_type=f32)
                    + b2_ref[...], 0.0)
    l = jnp.dot(z, w3_ref[...], preferred_element_type=f32) + b3_ref[...]
    m = jnp.max(l, axis=-1, keepdims=True)
    e = jnp.exp(l - m)
    o_ref[...] = e / jnp.sum(e, axis=-1, keepdims=True)


def _mlp(r, neigh, W_neigh, b_conv, W1, b1, W2, b2, W3, b3):
    B = 2000
    grid = (_N // B,)
    full = lambda shape: pl.BlockSpec(shape, lambda i: tuple(0 for _ in shape))
    return pl.pallas_call(
        _mlp_body,
        grid=grid,
        in_specs=[
            pl.BlockSpec((B, _H), lambda i: (i, 0)),
            pl.BlockSpec((2, B, 128), lambda i: (0, i, 0)),
            full((_D, _H)),
            full((1, _H)),
            full((_H, 400)),
            full((1, 400)),
            full((400, 400)),
            full((1, 400)),
            full((400, 2)),
            full((1, 2)),
        ],
        out_specs=pl.BlockSpec((B, 2), lambda i: (i, 0)),
        out_shape=jax.ShapeDtypeStruct((_N, 2), jnp.float32),
    )(r, neigh, W_neigh, b_conv.reshape(1, _H), W1,
      b1.reshape(1, 400), W2, b2.reshape(1, 400), W3, b3.reshape(1, 2))


@jax.jit
def kernel(x, edge_index, W_root, W_neigh, b_conv, W1, b1, W2, b2, W3, b3):
    xv = x.reshape(2 * _N, 128)
    pad = _EPAD - _E
    srcp = jnp.concatenate(
        [edge_index[0] * 2, jnp.zeros((pad,), jnp.int32)]).reshape(
            _NSUB, _NCH, _CHUNK)
    dstp = jnp.concatenate(
        [edge_index[1], jnp.full((pad,), _N, jnp.int32)]).reshape(
            _NSUB, _NCH, _CHUNK)
    idxp = jnp.stack([srcp, dstp], axis=2)
    neigh = _sc_aggregate(xv, idxp)
    r = _mm1(x, W_root)
    return _mlp(r, neigh, W_neigh, b_conv, W1, b1, W2, b2, W3, b3)


# speedup vs baseline: 1.8400x; 1.8400x over previous
import functools

import jax
import jax.numpy as jnp
from jax import lax
from jax.experimental import pallas as pl
from jax.experimental.pallas import tpu as pltpu
from jax.experimental.pallas import tpu_sc as plsc

_N = 10000
_E = 160000
_D = 256
_H = 512

_NSUB = 16
_CHUNK = 128
_NCH = 80
_EPS = _NCH * _CHUNK
_EPAD = _NSUB * _EPS
_NA = _N + 256
_RPS = _N // _NSUB


def _sc_aggregate(xv, idxp):
    mesh = plsc.VectorSubcoreMesh(core_axis_name="c", subcore_axis_name="s")

    @functools.partial(
        pl.kernel,
        out_type=jax.ShapeDtypeStruct((2, _N, 128), jnp.float32),
        mesh=mesh,
        scratch_types=[
            pltpu.VMEM((2, _CHUNK), jnp.int32),
            pltpu.VMEM((2, _CHUNK), jnp.int32),
            pltpu.VMEM((_CHUNK, 128), jnp.float32),
            pltpu.VMEM((_CHUNK, 128), jnp.float32),
            pltpu.VMEM((_CHUNK, 16), jnp.float32),
            pltpu.VMEM((_CHUNK, 16), jnp.float32),
            pltpu.VMEM_SHARED((_NA, 128), jnp.float32),
            pltpu.VMEM_SHARED((_NA, 16), jnp.float32),
            pltpu.SemaphoreType.DMA,
            pltpu.SemaphoreType.DMA,
        ],
        compiler_params=pltpu.CompilerParams(use_tc_tiling_on_sc=False),
    )
    def k(xv_hbm, idx_hbm, out_hbm, idx_a, idx_b, buf_a, buf_b, ones_v,
          deg_v, acc, dacc, sem_g, sem_i):
        c = lax.axis_index("c")
        s = lax.axis_index("s")
        idx = idx_hbm.at[s]
        cvec = jnp.full((16,), 0, jnp.int32) + c * _N

        zv = jnp.zeros((1, 16), jnp.float32)
        ov = jnp.ones((1, 16), jnp.float32)

        @pl.loop(0, _CHUNK)
        def _(r):
            for cc in range(0, 128, 16):
                buf_a[pl.ds(r, 1), pl.ds(cc, 16)] = zv
            ones_v[pl.ds(r, 1), pl.ds(0, 16)] = ov
            deg_v[pl.ds(r, 1), pl.ds(0, 16)] = zv

        @pl.loop(0, 6)
        def _(kk):
            ch = s + _NSUB * kk

            @pl.when(ch < _NA // _CHUNK)
            def _():
                pltpu.sync_copy(buf_a, acc.at[pl.ds(ch * _CHUNK, _CHUNK)])
                pltpu.sync_copy(deg_v, dacc.at[pl.ds(ch * _CHUNK, _CHUNK)])

            @pl.when(ch == _NA // _CHUNK)
            def _():
                r0 = ch * _CHUNK
                nr = _NA - r0
                pltpu.sync_copy(buf_a.at[pl.ds(0, nr)], acc.at[pl.ds(r0, nr)])
                pltpu.sync_copy(deg_v.at[pl.ds(0, nr)], dacc.at[pl.ds(r0, nr)])

        plsc.subcore_barrier()

        def load_idx_start(g, ib):
            pltpu.async_copy(idx.at[pl.ds(2 * g, 2)], ib, sem_i)

        def load_idx_wait(g, ib):
            pltpu.make_async_copy(idx.at[pl.ds(2 * g, 2)], ib, sem_i).wait()
            for cc in range(0, _CHUNK, 16):
                ib[pl.ds(0, 1), pl.ds(cc, 16)] = (
                    ib[pl.ds(0, 1), pl.ds(cc, 16)] + cvec.reshape(1, 16))

        load_idx_start(0, idx_a)
        load_idx_wait(0, idx_a)
        pltpu.async_copy(xv_hbm.at[idx_a.at[0]], buf_a, sem_g)
        load_idx_start(1, idx_b)

        @pl.loop(0, _NCH // 2)
        def _(g2):
            g = 2 * g2
            load_idx_wait(g + 1, idx_b)
            pltpu.make_async_copy(xv_hbm.at[idx_a.at[0]], buf_a, sem_g).wait()
            pltpu.async_copy(xv_hbm.at[idx_b.at[0]], buf_b, sem_g)
            pltpu.sync_copy(buf_a, acc.at[idx_a.at[1]], add=True)
            pltpu.sync_copy(ones_v, dacc.at[idx_a.at[1]], add=True)

            @pl.when(g2 < _NCH // 2 - 1)
            def _():
                load_idx_start(g + 2, idx_a)
                pltpu.make_async_copy(xv_hbm.at[idx_b.at[0]], buf_b, sem_g).wait()
                load_idx_wait(g + 2, idx_a)
                pltpu.async_copy(xv_hbm.at[idx_a.at[0]], buf_a, sem_g)
                pltpu.sync_copy(buf_b, acc.at[idx_b.at[1]], add=True)
                pltpu.sync_copy(ones_v, dacc.at[idx_b.at[1]], add=True)
                load_idx_start(g + 3, idx_b)

            @pl.when(g2 == _NCH // 2 - 1)
            def _():
                pltpu.make_async_copy(xv_hbm.at[idx_b.at[0]], buf_b, sem_g).wait()
                pltpu.sync_copy(buf_b, acc.at[idx_b.at[1]], add=True)
                pltpu.sync_copy(ones_v, dacc.at[idx_b.at[1]], add=True)

        plsc.subcore_barrier()

        @pl.loop(0, 5)
        def _(kk):
            r0 = s * _RPS + kk * 125
            pltpu.sync_copy(acc.at[pl.ds(r0, 125)], buf_a.at[pl.ds(0, 125)])
            pltpu.sync_copy(dacc.at[pl.ds(r0, 125)], deg_v.at[pl.ds(0, 125)])

            @pl.loop(0, 125)
            def _(r):
                ivec = 1.0 / jnp.maximum(deg_v[pl.ds(r, 1), pl.ds(0, 16)], 1.0)

                for cc in range(0, 128, 16):
                    buf_a[pl.ds(r, 1), pl.ds(cc, 16)] = (
                        buf_a[pl.ds(r, 1), pl.ds(cc, 16)] * ivec)

            pltpu.sync_copy(buf_a.at[pl.ds(0, 125)],
                            out_hbm.at[c].at[pl.ds(r0, 125)])

    return k(xv, idxp)


def _mm1_body(x_ref, wr_ref, o_ref):
    o_ref[...] = jnp.dot(x_ref[...], wr_ref[...],
                         preferred_element_type=jnp.float32)


def _mm1(x, W_root):
    B = 2000
    return pl.pallas_call(
        _mm1_body,
        grid=(_N // B,),
        in_specs=[
            pl.BlockSpec((B, _D), lambda i: (i, 0)),
            pl.BlockSpec((_D, _H), lambda i: (0, 0)),
        ],
        out_specs=pl.BlockSpec((B, _H), lambda i: (i, 0)),
        out_shape=jax.ShapeDtypeStruct((_N, _H), jnp.float32),
    )(x, W_root)


def _mlp_body(r_ref, nb_ref, wn_ref, bc_ref, w1_ref, b1_ref,
              w2_ref, b2_ref, w3_ref, b3_ref, o_ref):
    f32 = jnp.float32
    h = r_ref[...]
    h = h + jnp.dot(nb_ref[0], wn_ref[:128], preferred_element_type=f32)
    h = h + jnp.dot(nb_ref[1], wn_ref[128:], preferred_element_type=f32)
    h = jnp.maximum(h + bc_ref[...], 0.0)
    z = jnp.maximum(jnp.dot(h, w1_ref[...], preferred_element_type=f32)
                    + b1_ref[...], 0.0)
    z = jnp.maximum(jnp.dot(z, w2_ref[...], preferred_element_type=f32)
                    + b2_ref[...], 0.0)
    l = jnp.dot(z, w3_ref[...], preferred_element_type=f32) + b3_ref[...]
    m = jnp.max(l, axis=-1, keepdims=True)
    e = jnp.exp(l - m)
    o_ref[...] = e / jnp.sum(e, axis=-1, keepdims=True)


def _mlp(r, neigh, W_neigh, b_conv, W1, b1, W2, b2, W3, b3):
    B = 2000
    grid = (_N // B,)
    full = lambda shape: pl.BlockSpec(shape, lambda i: tuple(0 for _ in shape))
    return pl.pallas_call(
        _mlp_body,
        grid=grid,
        in_specs=[
            pl.BlockSpec((B, _H), lambda i: (i, 0)),
            pl.BlockSpec((2, B, 128), lambda i: (0, i, 0)),
            full((_D, _H)),
            full((1, _H)),
            full((_H, 400)),
            full((1, 400)),
            full((400, 400)),
            full((1, 400)),
            full((400, 2)),
            full((1, 2)),
        ],
        out_specs=pl.BlockSpec((B, 2), lambda i: (i, 0)),
        out_shape=jax.ShapeDtypeStruct((_N, 2), jnp.float32),
    )(r, neigh, W_neigh, b_conv.reshape(1, _H), W1,
      b1.reshape(1, 400), W2, b2.reshape(1, 400), W3, b3.reshape(1, 2))


@jax.jit
def kernel(x, edge_index, W_root, W_neigh, b_conv, W1, b1, W2, b2, W3, b3):
    xv = jnp.concatenate([x[:, :128], x[:, 128:]], axis=0)
    pad = _EPAD - _E
    padi = jnp.arange(pad, dtype=jnp.int32)
    srcp = jnp.concatenate(
        [edge_index[0], padi % _N]).reshape(_NSUB, _NCH, _CHUNK)
    dstp = jnp.concatenate(
        [edge_index[1], _N + (padi % 256)]).reshape(_NSUB, _NCH, _CHUNK)
    idxp = jnp.stack([srcp, dstp], axis=2).reshape(_NSUB, 2 * _NCH, _CHUNK)
    neigh = _sc_aggregate(xv, idxp)
    r = _mm1(x, W_root)
    return _mlp(r, neigh, W_neigh, b_conv, W1, b1, W2, b2, W3, b3)


# speedup vs baseline: 1.8544x; 1.0078x over previous
import functools

import jax
import jax.numpy as jnp
from jax import lax
from jax.experimental import pallas as pl
from jax.experimental.pallas import tpu as pltpu
from jax.experimental.pallas import tpu_sc as plsc

_N = 10000
_E = 160000
_D = 256
_H = 512

_NSUB = 16
_CHUNK = 128
_NCH = 80
_EPS = _NCH * _CHUNK
_EPAD = _NSUB * _EPS
_NA = _N + 256
_RPS = _N // _NSUB


def _sc_aggregate(xv, idxp):
    mesh = plsc.VectorSubcoreMesh(core_axis_name="c", subcore_axis_name="s")

    @functools.partial(
        pl.kernel,
        out_type=jax.ShapeDtypeStruct((2, _N, 128), jnp.float32),
        mesh=mesh,
        scratch_types=[
            pltpu.VMEM((8, _CHUNK), jnp.int32),
            pltpu.VMEM((_CHUNK, 128), jnp.float32),
            pltpu.VMEM((_CHUNK, 128), jnp.float32),
            pltpu.VMEM((_CHUNK, 16), jnp.float32),
            pltpu.VMEM((_CHUNK, 16), jnp.float32),
            pltpu.VMEM_SHARED((_NA, 128), jnp.float32),
            pltpu.VMEM_SHARED((_NA, 16), jnp.float32),
            pltpu.SemaphoreType.DMA,
            pltpu.SemaphoreType.DMA,
            pltpu.SemaphoreType.DMA,
        ],
        compiler_params=pltpu.CompilerParams(use_tc_tiling_on_sc=False),
    )
    def k(xv_hbm, idx_hbm, out_hbm, idx_v, buf_a, buf_b, ones_v,
          deg_v, acc, dacc, sem_g, sem_i, sem_s):
        c = lax.axis_index("c")
        s = lax.axis_index("s")
        idx = idx_hbm.at[s]
        cvec = jnp.full((16,), 0, jnp.int32) + c * _N

        zv = jnp.zeros((1, 16), jnp.float32)
        ov = jnp.ones((1, 16), jnp.float32)

        @pl.loop(0, _CHUNK)
        def _(r):
            for cc in range(0, 128, 16):
                buf_a[pl.ds(r, 1), pl.ds(cc, 16)] = zv
            ones_v[pl.ds(r, 1), pl.ds(0, 16)] = ov
            deg_v[pl.ds(r, 1), pl.ds(0, 16)] = zv

        @pl.loop(0, 6)
        def _(kk):
            ch = s + _NSUB * kk

            @pl.when(ch < _NA // _CHUNK)
            def _():
                pltpu.sync_copy(buf_a, acc.at[pl.ds(ch * _CHUNK, _CHUNK)])
                pltpu.sync_copy(deg_v, dacc.at[pl.ds(ch * _CHUNK, _CHUNK)])

            @pl.when(ch == _NA // _CHUNK)
            def _():
                r0 = ch * _CHUNK
                nr = _NA - r0
                pltpu.sync_copy(buf_a.at[pl.ds(0, nr)], acc.at[pl.ds(r0, nr)])
                pltpu.sync_copy(deg_v.at[pl.ds(0, nr)], dacc.at[pl.ds(r0, nr)])

        plsc.subcore_barrier()

        def idx_start(g, sl):
            pltpu.async_copy(idx.at[pl.ds(2 * g, 2)], idx_v.at[pl.ds(2 * sl, 2)], sem_i)

        def idx_wait(g, sl):
            pltpu.make_async_copy(idx.at[pl.ds(2 * g, 2)],
                                  idx_v.at[pl.ds(2 * sl, 2)], sem_i).wait()
            for cc in range(0, _CHUNK, 16):
                idx_v[pl.ds(2 * sl, 1), pl.ds(cc, 16)] = (
                    idx_v[pl.ds(2 * sl, 1), pl.ds(cc, 16)] + cvec.reshape(1, 16))

        def gather_start(sl, buf):
            pltpu.async_copy(xv_hbm.at[idx_v.at[2 * sl]], buf, sem_g)

        def gather_wait(sl, buf):
            pltpu.make_async_copy(xv_hbm.at[idx_v.at[2 * sl]], buf, sem_g).wait()

        def scatter_start(sl, buf):
            pltpu.async_copy(buf, acc.at[idx_v.at[2 * sl + 1]], sem_s, add=True)
            pltpu.async_copy(ones_v, dacc.at[idx_v.at[2 * sl + 1]], sem_s, add=True)

        def scatter_wait(sl, buf):
            pltpu.make_async_copy(buf, acc.at[idx_v.at[2 * sl + 1]], sem_s).wait()
            pltpu.make_async_copy(ones_v, dacc.at[idx_v.at[2 * sl + 1]], sem_s).wait()

        idx_start(0, 0)
        idx_wait(0, 0)
        gather_start(0, buf_a)
        idx_start(1, 1)
        idx_start(2, 2)

        @pl.loop(0, _NCH // 4)
        def _(b):
            for j in range(4):
                p, q = (buf_a, buf_b) if j % 2 == 0 else (buf_b, buf_a)
                k = 4 * b + j

                @pl.when(k < _NCH - 1)
                def _():
                    idx_wait(k + 1, (j + 1) % 4)

                gather_wait(j % 4, p)

                @pl.when(k > 0)
                def _():
                    scatter_wait((j + 3) % 4, q)

                @pl.when(k < _NCH - 1)
                def _():
                    gather_start((j + 1) % 4, q)

                scatter_start(j % 4, p)

                @pl.when(k + 3 < _NCH)
                def _():
                    idx_start(k + 3, (j + 3) % 4)

        scatter_wait(3, buf_b)

        plsc.subcore_barrier()

        @pl.loop(0, 5)
        def _(kk):
            r0 = s * _RPS + kk * 125
            pltpu.sync_copy(acc.at[pl.ds(r0, 125)], buf_a.at[pl.ds(0, 125)])
            pltpu.sync_copy(dacc.at[pl.ds(r0, 125)], deg_v.at[pl.ds(0, 125)])

            @pl.loop(0, 125)
            def _(r):
                ivec = 1.0 / jnp.maximum(deg_v[pl.ds(r, 1), pl.ds(0, 16)], 1.0)

                for cc in range(0, 128, 16):
                    buf_a[pl.ds(r, 1), pl.ds(cc, 16)] = (
                        buf_a[pl.ds(r, 1), pl.ds(cc, 16)] * ivec)

            pltpu.sync_copy(buf_a.at[pl.ds(0, 125)],
                            out_hbm.at[c].at[pl.ds(r0, 125)])

    return k(xv, idxp)


def _mm1_body(x_ref, wr_ref, o_ref):
    o_ref[...] = jnp.dot(x_ref[...], wr_ref[...],
                         preferred_element_type=jnp.float32)


def _mm1(x, W_root):
    B = 2000
    return pl.pallas_call(
        _mm1_body,
        grid=(_N // B,),
        in_specs=[
            pl.BlockSpec((B, _D), lambda i: (i, 0)),
            pl.BlockSpec((_D, _H), lambda i: (0, 0)),
        ],
        out_specs=pl.BlockSpec((B, _H), lambda i: (i, 0)),
        out_shape=jax.ShapeDtypeStruct((_N, _H), jnp.float32),
    )(x, W_root)


def _mlp_body(r_ref, nb_ref, wn_ref, bc_ref, w1_ref, b1_ref,
              w2_ref, b2_ref, w3_ref, b3_ref, o_ref):
    f32 = jnp.float32
    h = r_ref[...]
    h = h + jnp.dot(nb_ref[0], wn_ref[:128], preferred_element_type=f32)
    h = h + jnp.dot(nb_ref[1], wn_ref[128:], preferred_element_type=f32)
    h = jnp.maximum(h + bc_ref[...], 0.0)
    z = jnp.maximum(jnp.dot(h, w1_ref[...], preferred_element_type=f32)
                    + b1_ref[...], 0.0)
    z = jnp.maximum(jnp.dot(z, w2_ref[...], preferred_element_type=f32)
                    + b2_ref[...], 0.0)
    l = jnp.dot(z, w3_ref[...], preferred_element_type=f32) + b3_ref[...]
    m = jnp.max(l, axis=-1, keepdims=True)
    e = jnp.exp(l - m)
    o_ref[...] = e / jnp.sum(e, axis=-1, keepdims=True)


def _mlp(r, neigh, W_neigh, b_conv, W1, b1, W2, b2, W3, b3):
    B = 2000
    grid = (_N // B,)
    full = lambda shape: pl.BlockSpec(shape, lambda i: tuple(0 for _ in shape))
    return pl.pallas_call(
        _mlp_body,
        grid=grid,
        in_specs=[
            pl.BlockSpec((B, _H), lambda i: (i, 0)),
            pl.BlockSpec((2, B, 128), lambda i: (0, i, 0)),
            full((_D, _H)),
            full((1, _H)),
            full((_H, 400)),
            full((1, 400)),
            full((400, 400)),
            full((1, 400)),
            full((400, 2)),
            full((1, 2)),
        ],
        out_specs=pl.BlockSpec((B, 2), lambda i: (i, 0)),
        out_shape=jax.ShapeDtypeStruct((_N, 2), jnp.float32),
    )(r, neigh, W_neigh, b_conv.reshape(1, _H), W1,
      b1.reshape(1, 400), W2, b2.reshape(1, 400), W3, b3.reshape(1, 2))


@jax.jit
def kernel(x, edge_index, W_root, W_neigh, b_conv, W1, b1, W2, b2, W3, b3):
    xv = jnp.concatenate([x[:, :128], x[:, 128:]], axis=0)
    pad = _EPAD - _E
    padi = jnp.arange(pad, dtype=jnp.int32)
    srcp = jnp.concatenate(
        [edge_index[0], padi % _N]).reshape(_NSUB, _NCH, _CHUNK)
    dstp = jnp.concatenate(
        [edge_index[1], _N + (padi % 256)]).reshape(_NSUB, _NCH, _CHUNK)
    idxp = jnp.stack([srcp, dstp], axis=2).reshape(_NSUB, 2 * _NCH, _CHUNK)
    neigh = _sc_aggregate(xv, idxp)
    r = _mm1(x, W_root)
    return _mlp(r, neigh, W_neigh, b_conv, W1, b1, W2, b2, W3, b3)


# speedup vs baseline: 1.8806x; 1.0141x over previous
import functools

import jax
import jax.numpy as jnp
from jax import lax
from jax.experimental import pallas as pl
from jax.experimental.pallas import tpu as pltpu
from jax.experimental.pallas import tpu_sc as plsc

_N = 10000
_E = 160000
_D = 256
_H = 512

_NSUB = 16
_CHUNK = 128
_NCH = 80
_EPS = _NCH * _CHUNK
_EPAD = _NSUB * _EPS
_NA = _N + 256
_RPS = _N // _NSUB


def _sc_aggregate(xv, srcp, dstp):
    mesh = plsc.VectorSubcoreMesh(core_axis_name="c", subcore_axis_name="s")

    @functools.partial(
        pl.kernel,
        out_type=jax.ShapeDtypeStruct((2, _N, 128), jnp.float32),
        mesh=mesh,
        scratch_types=[
            pltpu.VMEM((8, _CHUNK), jnp.int32),
            pltpu.VMEM((_CHUNK, 128), jnp.float32),
            pltpu.VMEM((_CHUNK, 128), jnp.float32),
            pltpu.VMEM((_CHUNK, 16), jnp.float32),
            pltpu.VMEM((_CHUNK, 16), jnp.float32),
            pltpu.VMEM_SHARED((_NA, 128), jnp.float32),
            pltpu.VMEM_SHARED((_NA, 16), jnp.float32),
            pltpu.SemaphoreType.DMA,
            pltpu.SemaphoreType.DMA,
            pltpu.SemaphoreType.DMA,
        ],
        compiler_params=pltpu.CompilerParams(use_tc_tiling_on_sc=False),
    )
    def k(xv_hbm, src_hbm, dst_hbm, out_hbm, idx_v, buf_a, buf_b, ones_v,
          deg_v, acc, dacc, sem_g, sem_i, sem_s):
        c = lax.axis_index("c")
        s = lax.axis_index("s")
        srcr = src_hbm.at[s]
        dstr = dst_hbm.at[s]
        cvec = jnp.full((16,), 0, jnp.int32) + c * _N

        zv = jnp.zeros((1, 16), jnp.float32)
        ov = jnp.ones((1, 16), jnp.float32)

        @pl.loop(0, _CHUNK)
        def _(r):
            for cc in range(0, 128, 16):
                buf_a[pl.ds(r, 1), pl.ds(cc, 16)] = zv
            ones_v[pl.ds(r, 1), pl.ds(0, 16)] = ov
            deg_v[pl.ds(r, 1), pl.ds(0, 16)] = zv

        @pl.loop(0, 6)
        def _(kk):
            ch = s + _NSUB * kk

            @pl.when(ch < _NA // _CHUNK)
            def _():
                pltpu.sync_copy(buf_a, acc.at[pl.ds(ch * _CHUNK, _CHUNK)])
                pltpu.sync_copy(deg_v, dacc.at[pl.ds(ch * _CHUNK, _CHUNK)])

            @pl.when(ch == _NA // _CHUNK)
            def _():
                r0 = ch * _CHUNK
                nr = _NA - r0
                pltpu.sync_copy(buf_a.at[pl.ds(0, nr)], acc.at[pl.ds(r0, nr)])
                pltpu.sync_copy(deg_v.at[pl.ds(0, nr)], dacc.at[pl.ds(r0, nr)])

        plsc.subcore_barrier()

        def idx_start(g, sl):
            pltpu.async_copy(srcr.at[g], idx_v.at[2 * sl], sem_i)
            pltpu.async_copy(dstr.at[g], idx_v.at[2 * sl + 1], sem_i)

        def idx_wait(g, sl):
            pltpu.make_async_copy(srcr.at[g], idx_v.at[2 * sl], sem_i).wait()
            pltpu.make_async_copy(dstr.at[g], idx_v.at[2 * sl + 1], sem_i).wait()
            for cc in range(0, _CHUNK, 16):
                idx_v[pl.ds(2 * sl, 1), pl.ds(cc, 16)] = (
                    idx_v[pl.ds(2 * sl, 1), pl.ds(cc, 16)] + cvec.reshape(1, 16))

        def gather_start(sl, buf):
            pltpu.async_copy(xv_hbm.at[idx_v.at[2 * sl]], buf, sem_g)

        def gather_wait(sl, buf):
            pltpu.make_async_copy(xv_hbm.at[idx_v.at[2 * sl]], buf, sem_g).wait()

        def scatter_start(sl, buf):
            pltpu.async_copy(buf, acc.at[idx_v.at[2 * sl + 1]], sem_s, add=True)
            pltpu.async_copy(ones_v, dacc.at[idx_v.at[2 * sl + 1]], sem_s, add=True)

        def scatter_wait(sl, buf):
            pltpu.make_async_copy(buf, acc.at[idx_v.at[2 * sl + 1]], sem_s).wait()
            pltpu.make_async_copy(ones_v, dacc.at[idx_v.at[2 * sl + 1]], sem_s).wait()

        idx_start(0, 0)
        idx_wait(0, 0)
        gather_start(0, buf_a)
        idx_start(1, 1)
        idx_start(2, 2)

        @pl.loop(0, _NCH // 4)
        def _(b):
            for j in range(4):
                p, q = (buf_a, buf_b) if j % 2 == 0 else (buf_b, buf_a)
                k = 4 * b + j

                @pl.when(k < _NCH - 1)
                def _():
                    idx_wait(k + 1, (j + 1) % 4)

                gather_wait(j % 4, p)

                @pl.when(k > 0)
                def _():
                    scatter_wait((j + 3) % 4, q)

                @pl.when(k < _NCH - 1)
                def _():
                    gather_start((j + 1) % 4, q)

                scatter_start(j % 4, p)

                @pl.when(k + 3 < _NCH)
                def _():
                    idx_start(k + 3, (j + 3) % 4)

        scatter_wait(3, buf_b)

        plsc.subcore_barrier()

        @pl.loop(0, 5)
        def _(kk):
            r0 = s * _RPS + kk * 125
            pltpu.sync_copy(acc.at[pl.ds(r0, 125)], buf_a.at[pl.ds(0, 125)])
            pltpu.sync_copy(dacc.at[pl.ds(r0, 125)], deg_v.at[pl.ds(0, 125)])

            @pl.loop(0, 125)
            def _(r):
                ivec = 1.0 / jnp.maximum(deg_v[pl.ds(r, 1), pl.ds(0, 16)], 1.0)

                for cc in range(0, 128, 16):
                    buf_a[pl.ds(r, 1), pl.ds(cc, 16)] = (
                        buf_a[pl.ds(r, 1), pl.ds(cc, 16)] * ivec)

            pltpu.sync_copy(buf_a.at[pl.ds(0, 125)],
                            out_hbm.at[c].at[pl.ds(r0, 125)])

    return k(xv, srcp, dstp)


def _xv_body(x_ref, o_ref):
    o_ref[...] = x_ref[...]


def _xv(x):
    B = 2000
    return pl.pallas_call(
        _xv_body,
        grid=(2, _N // B),
        in_specs=[pl.BlockSpec((B, 128), lambda c, i: (i, c))],
        out_specs=pl.BlockSpec((B, 128), lambda c, i: (c * (_N // B) + i, 0)),
        out_shape=jax.ShapeDtypeStruct((2 * _N, 128), jnp.float32),
    )(x)


def _mm1_body(x_ref, wr_ref, o_ref):
    o_ref[...] = jnp.dot(x_ref[...], wr_ref[...],
                         preferred_element_type=jnp.float32)


def _mm1(x, W_root):
    B = 2000
    return pl.pallas_call(
        _mm1_body,
        grid=(_N // B,),
        in_specs=[
            pl.BlockSpec((B, _D), lambda i: (i, 0)),
            pl.BlockSpec((_D, _H), lambda i: (0, 0)),
        ],
        out_specs=pl.BlockSpec((B, _H), lambda i: (i, 0)),
        out_shape=jax.ShapeDtypeStruct((_N, _H), jnp.float32),
    )(x, W_root)


def _mlp_body(r_ref, nb_ref, wn_ref, bc_ref, w1_ref, b1_ref,
              w2_ref, b2_ref, w3_ref, b3_ref, o_ref):
    f32 = jnp.float32
    h = r_ref[...]
    h = h + jnp.dot(nb_ref[0], wn_ref[:128], preferred_element_type=f32)
    h = h + jnp.dot(nb_ref[1], wn_ref[128:], preferred_element_type=f32)
    h = jnp.maximum(h + bc_ref[...], 0.0)
    z = jnp.maximum(jnp.dot(h, w1_ref[...], preferred_element_type=f32)
                    + b1_ref[...], 0.0)
    z = jnp.maximum(jnp.dot(z, w2_ref[...], preferred_element_type=f32)
                    + b2_ref[...], 0.0)
    l = jnp.dot(z, w3_ref[...], preferred_element_type=f32) + b3_ref[...]
    m = jnp.max(l, axis=-1, keepdims=True)
    e = jnp.exp(l - m)
    o_ref[...] = e / jnp.sum(e, axis=-1, keepdims=True)


def _mlp(r, neigh, W_neigh, b_conv, W1, b1, W2, b2, W3, b3):
    B = 2000
    grid = (_N // B,)
    full = lambda shape: pl.BlockSpec(shape, lambda i: tuple(0 for _ in shape))
    return pl.pallas_call(
        _mlp_body,
        grid=grid,
        in_specs=[
            pl.BlockSpec((B, _H), lambda i: (i, 0)),
            pl.BlockSpec((2, B, 128), lambda i: (0, i, 0)),
            full((_D, _H)),
            full((1, _H)),
            full((_H, 400)),
            full((1, 400)),
            full((400, 400)),
            full((1, 400)),
            full((400, 2)),
            full((1, 2)),
        ],
        out_specs=pl.BlockSpec((B, 2), lambda i: (i, 0)),
        out_shape=jax.ShapeDtypeStruct((_N, 2), jnp.float32),
    )(r, neigh, W_neigh, b_conv.reshape(1, _H), W1,
      b1.reshape(1, 400), W2, b2.reshape(1, 400), W3, b3.reshape(1, 2))


@jax.jit
def kernel(x, edge_index, W_root, W_neigh, b_conv, W1, b1, W2, b2, W3, b3):
    xv = _xv(x)
    pad = _EPAD - _E
    padi = jnp.arange(pad, dtype=jnp.int32)
    srcp = jnp.concatenate(
        [edge_index[0], padi % _N]).reshape(_NSUB, _NCH, _CHUNK)
    dstp = jnp.concatenate(
        [edge_index[1], _N + (padi % 256)]).reshape(_NSUB, _NCH, _CHUNK)
    neigh = _sc_aggregate(xv, srcp, dstp)
    r = _mm1(x, W_root)
    return _mlp(r, neigh, W_neigh, b_conv, W1, b1, W2, b2, W3, b3)


# speedup vs baseline: 1.9289x; 1.0257x over previous
import functools

import jax
import jax.numpy as jnp
from jax import lax
from jax.experimental import pallas as pl
from jax.experimental.pallas import tpu as pltpu
from jax.experimental.pallas import tpu_sc as plsc

_N = 10000
_E = 160000
_D = 256
_H = 512

_NSUB = 16
_CHUNK = 128
_NCH = 80
_EPS = _NCH * _CHUNK
_EPAD = _NSUB * _EPS
_NA = _N + 256
_RPS = _N // _NSUB


def _sc_aggregate(xv, edges):
    mesh = plsc.VectorSubcoreMesh(core_axis_name="c", subcore_axis_name="s")

    @functools.partial(
        pl.kernel,
        out_type=jax.ShapeDtypeStruct((2, _N, 128), jnp.float32),
        mesh=mesh,
        scratch_types=[
            pltpu.VMEM((8, _CHUNK), jnp.int32),
            pltpu.VMEM((_CHUNK, 128), jnp.float32),
            pltpu.VMEM((_CHUNK, 128), jnp.float32),
            pltpu.VMEM((_CHUNK, 16), jnp.float32),
            pltpu.VMEM((_CHUNK, 16), jnp.float32),
            pltpu.VMEM_SHARED((_NA, 128), jnp.float32),
            pltpu.VMEM_SHARED((_NA, 16), jnp.float32),
            pltpu.SemaphoreType.DMA,
            pltpu.SemaphoreType.DMA,
            pltpu.SemaphoreType.DMA,
        ],
        compiler_params=pltpu.CompilerParams(use_tc_tiling_on_sc=False),
    )
    def k(xv_hbm, e_hbm, out_hbm, idx_v, buf_a, buf_b, ones_v,
          deg_v, acc, dacc, sem_g, sem_i, sem_s):
        c = lax.axis_index("c")
        s = lax.axis_index("s")
        srcr = e_hbm.at[0].at[s]
        dstr = e_hbm.at[1].at[s]
        cvec = jnp.full((16,), 0, jnp.int32) + c * _N

        zv = jnp.zeros((1, 16), jnp.float32)
        ov = jnp.ones((1, 16), jnp.float32)

        @pl.loop(0, _CHUNK)
        def _(r):
            for cc in range(0, 128, 16):
                buf_a[pl.ds(r, 1), pl.ds(cc, 16)] = zv
            ones_v[pl.ds(r, 1), pl.ds(0, 16)] = ov
            deg_v[pl.ds(r, 1), pl.ds(0, 16)] = zv

        @pl.loop(0, 6)
        def _(kk):
            ch = s + _NSUB * kk

            @pl.when(ch < _NA // _CHUNK)
            def _():
                pltpu.sync_copy(buf_a, acc.at[pl.ds(ch * _CHUNK, _CHUNK)])
                pltpu.sync_copy(deg_v, dacc.at[pl.ds(ch * _CHUNK, _CHUNK)])

            @pl.when(ch == _NA // _CHUNK)
            def _():
                r0 = ch * _CHUNK
                nr = _NA - r0
                pltpu.sync_copy(buf_a.at[pl.ds(0, nr)], acc.at[pl.ds(r0, nr)])
                pltpu.sync_copy(deg_v.at[pl.ds(0, nr)], dacc.at[pl.ds(r0, nr)])

        plsc.subcore_barrier()

        def idx_start(g, sl):
            pltpu.async_copy(srcr.at[g], idx_v.at[2 * sl], sem_i)
            pltpu.async_copy(dstr.at[g], idx_v.at[2 * sl + 1], sem_i)

        def idx_wait(g, sl):
            pltpu.make_async_copy(srcr.at[g], idx_v.at[2 * sl], sem_i).wait()
            pltpu.make_async_copy(dstr.at[g], idx_v.at[2 * sl + 1], sem_i).wait()
            for cc in range(0, _CHUNK, 16):
                idx_v[pl.ds(2 * sl, 1), pl.ds(cc, 16)] = (
                    idx_v[pl.ds(2 * sl, 1), pl.ds(cc, 16)] + cvec.reshape(1, 16))

        def gather_start(sl, buf):
            pltpu.async_copy(xv_hbm.at[idx_v.at[2 * sl]], buf, sem_g)

        def gather_wait(sl, buf):
            pltpu.make_async_copy(xv_hbm.at[idx_v.at[2 * sl]], buf, sem_g).wait()

        def scatter_start(sl, buf):
            pltpu.async_copy(buf, acc.at[idx_v.at[2 * sl + 1]], sem_s, add=True)
            pltpu.async_copy(ones_v, dacc.at[idx_v.at[2 * sl + 1]], sem_s, add=True)

        def scatter_wait(sl, buf):
            pltpu.make_async_copy(buf, acc.at[idx_v.at[2 * sl + 1]], sem_s).wait()
            pltpu.make_async_copy(ones_v, dacc.at[idx_v.at[2 * sl + 1]], sem_s).wait()

        idx_start(0, 0)
        idx_wait(0, 0)
        gather_start(0, buf_a)
        idx_start(1, 1)
        idx_start(2, 2)

        @pl.loop(0, _NCH // 4)
        def _(b):
            for j in range(4):
                p, q = (buf_a, buf_b) if j % 2 == 0 else (buf_b, buf_a)
                k = 4 * b + j

                @pl.when(k < _NCH - 1)
                def _():
                    idx_wait(k + 1, (j + 1) % 4)

                gather_wait(j % 4, p)

                @pl.when(k > 0)
                def _():
                    scatter_wait((j + 3) % 4, q)

                @pl.when(k < _NCH - 1)
                def _():
                    gather_start((j + 1) % 4, q)

                scatter_start(j % 4, p)

                @pl.when(k + 3 < _NCH)
                def _():
                    idx_start(k + 3, (j + 3) % 4)

        scatter_wait(3, buf_b)

        plsc.subcore_barrier()

        @pl.loop(0, 5)
        def _(kk):
            r0 = s * _RPS + kk * 125
            pltpu.sync_copy(acc.at[pl.ds(r0, 125)], buf_a.at[pl.ds(0, 125)])
            pltpu.sync_copy(dacc.at[pl.ds(r0, 125)], deg_v.at[pl.ds(0, 125)])

            @pl.loop(0, 125)
            def _(r):
                ivec = 1.0 / jnp.maximum(deg_v[pl.ds(r, 1), pl.ds(0, 16)], 1.0)

                for cc in range(0, 128, 16):
                    buf_a[pl.ds(r, 1), pl.ds(cc, 16)] = (
                        buf_a[pl.ds(r, 1), pl.ds(cc, 16)] * ivec)

            pltpu.sync_copy(buf_a.at[pl.ds(0, 125)],
                            out_hbm.at[c].at[pl.ds(r0, 125)])

    return k(xv, edges)


def _xv_body(x_ref, o_ref):
    o_ref[...] = x_ref[...]


def _xv(x):
    B = 2000
    return pl.pallas_call(
        _xv_body,
        grid=(2, _N // B),
        in_specs=[pl.BlockSpec((B, 128), lambda c, i: (i, c))],
        out_specs=pl.BlockSpec((B, 128), lambda c, i: (c * (_N // B) + i, 0)),
        out_shape=jax.ShapeDtypeStruct((2 * _N, 128), jnp.float32),
    )(x)


def _mm1_body(x_ref, wr_ref, o_ref):
    o_ref[...] = jnp.dot(x_ref[...], wr_ref[...],
                         preferred_element_type=jnp.float32)


def _mm1(x, W_root):
    B = 2000
    return pl.pallas_call(
        _mm1_body,
        grid=(_N // B,),
        in_specs=[
            pl.BlockSpec((B, _D), lambda i: (i, 0)),
            pl.BlockSpec((_D, _H), lambda i: (0, 0)),
        ],
        out_specs=pl.BlockSpec((B, _H), lambda i: (i, 0)),
        out_shape=jax.ShapeDtypeStruct((_N, _H), jnp.float32),
    )(x, W_root)


def _mlp_body(r_ref, nb_ref, wn_ref, bc_ref, w1_ref, b1_ref,
              w2_ref, b2_ref, w3_ref, b3_ref, o_ref):
    f32 = jnp.float32
    h = r_ref[...]
    h = h + jnp.dot(nb_ref[0], wn_ref[:128], preferred_element_type=f32)
    h = h + jnp.dot(nb_ref[1], wn_ref[128:], preferred_element_type=f32)
    h = jnp.maximum(h + bc_ref[...], 0.0)
    z = jnp.maximum(jnp.dot(h, w1_ref[...], preferred_element_type=f32)
                    + b1_ref[...], 0.0)
    z = jnp.maximum(jnp.dot(z, w2_ref[...], preferred_element_type=f32)
                    + b2_ref[...], 0.0)
    l = jnp.dot(z, w3_ref[...], preferred_element_type=f32) + b3_ref[...]
    m = jnp.max(l, axis=-1, keepdims=True)
    e = jnp.exp(l - m)
    o_ref[...] = e / jnp.sum(e, axis=-1, keepdims=True)


def _mlp(r, neigh, W_neigh, b_conv, W1, b1, W2, b2, W3, b3):
    B = 2000
    grid = (_N // B,)
    full = lambda shape: pl.BlockSpec(shape, lambda i: tuple(0 for _ in shape))
    return pl.pallas_call(
        _mlp_body,
        grid=grid,
        in_specs=[
            pl.BlockSpec((B, _H), lambda i: (i, 0)),
            pl.BlockSpec((2, B, 128), lambda i: (0, i, 0)),
            full((_D, _H)),
            full((1, _H)),
            full((_H, 400)),
            full((1, 400)),
            full((400, 400)),
            full((1, 400)),
            full((400, 2)),
            full((1, 2)),
        ],
        out_specs=pl.BlockSpec((B, 2), lambda i: (i, 0)),
        out_shape=jax.ShapeDtypeStruct((_N, 2), jnp.float32),
    )(r, neigh, W_neigh, b_conv.reshape(1, _H), W1,
      b1.reshape(1, 400), W2, b2.reshape(1, 400), W3, b3.reshape(1, 2))


@jax.jit
def kernel(x, edge_index, W_root, W_neigh, b_conv, W1, b1, W2, b2, W3, b3):
    xv = _xv(x)
    pad = _EPAD - _E
    padi = jnp.arange(pad, dtype=jnp.int32)
    pads = jnp.stack([padi % _N, _N + (padi % 256)])
    edges = jnp.concatenate([edge_index, pads], axis=1).reshape(
        2, _NSUB, _NCH, _CHUNK)
    neigh = _sc_aggregate(xv, edges)
    r = _mm1(x, W_root)
    return _mlp(r, neigh, W_neigh, b_conv, W1, b1, W2, b2, W3, b3)
